# Initial kernel scaffold; baseline (speedup 1.0000x reference)
import jax
import jax.numpy as jnp
from jax.experimental import pallas as pl


def kernel(x, edge_index, W_gcn, b_gcn, W_lin, b_lin):
    raise NotImplementedError("write your pallas kernel here")


# speedup vs baseline: 9.2042x; 9.2042x over previous
import functools

import jax
import jax.numpy as jnp
from jax import lax
from jax.experimental import pallas as pl
from jax.experimental.pallas import tpu as pltpu
from jax.experimental.pallas import tpu_sc as plsc

N = 10000
E = 160000
D = 256
H = D // 2
NC = 2
NS = 16
N_PAD = 10240

_sc_mesh = plsc.VectorSubcoreMesh(core_axis_name="c", subcore_axis_name="s")

_DEG_K = 40
_DEG_EPT = E // (NC * NS)
_DEG_CHUNKS = _DEG_EPT // _DEG_K
_ZPT = N_PAD // NS


@functools.partial(
    pl.kernel,
    out_type=jax.ShapeDtypeStruct((NC * N_PAD,), jnp.float32),
    mesh=_sc_mesh,
    scratch_types=[
        pltpu.VMEM((48,), jnp.float32),
        pltpu.VMEM((_ZPT,), jnp.float32),
        pltpu.VMEM((_DEG_K,), jnp.int32),
        pltpu.VMEM_SHARED((N_PAD,), jnp.float32),
    ],
)
def _deg_kernel(dst_hbm, out_hbm, ones_v, zeros_v, didx_v, acc_s):
    c = lax.axis_index("c")
    s = lax.axis_index("s")
    for j in range(3):
        ones_v[pl.ds(j * 16, 16)] = jnp.ones((16,), jnp.float32)

    def zfill(i, carry):
        zeros_v[pl.ds(i * 16, 16)] = jnp.zeros((16,), jnp.float32)
        return carry

    lax.fori_loop(0, _ZPT // 16, zfill, 0)
    pltpu.sync_copy(zeros_v, acc_s.at[pl.ds(s * _ZPT, _ZPT)])
    plsc.subcore_barrier()

    ebase0 = c * (E // NC) + s * _DEG_EPT

    def chunk(g, carry):
        pltpu.sync_copy(dst_hbm.at[pl.ds(ebase0 + g * _DEG_K, _DEG_K)], didx_v)
        pltpu.sync_copy(ones_v.at[pl.ds(0, _DEG_K)], acc_s.at[didx_v], add=True)
        return carry

    lax.fori_loop(0, _DEG_CHUNKS, chunk, 0)
    plsc.subcore_barrier()
    pltpu.sync_copy(acc_s.at[pl.ds(s * _ZPT, _ZPT)],
                    out_hbm.at[pl.ds(c * N_PAD + s * _ZPT, _ZPT)])


_AGG_K = 80
_AGG_EPT = E // NS
_AGG_CHUNKS = _AGG_EPT // _AGG_K
_RPT = 640
_RSTRIDE = 624


@functools.partial(
    pl.kernel,
    out_type=jax.ShapeDtypeStruct((NC * N, H), jnp.float32),
    mesh=_sc_mesh,
    scratch_types=[
        pltpu.VMEM((_AGG_K,), jnp.int32),
        pltpu.VMEM((_AGG_K,), jnp.int32),
        pltpu.VMEM((_AGG_K, H), jnp.float32),
        pltpu.VMEM_SHARED((N, H), jnp.float32),
        pltpu.SemaphoreType.DMA,
    ],
)
def _agg_kernel(zcat_hbm, src_hbm, dst_hbm, out_hbm,
                sidx_v, didx_v, rows_v, acc_s, sem):
    c = lax.axis_index("c")
    s = lax.axis_index("s")
    row0 = s * _RSTRIDE
    pltpu.sync_copy(zcat_hbm.at[pl.ds(c * N + row0, _RPT)],
                    acc_s.at[pl.ds(row0, _RPT)])
    plsc.subcore_barrier()
    off = c * N

    def chunk(g, carry):
        ebase = s * _AGG_EPT + g * _AGG_K
        pltpu.sync_copy(src_hbm.at[pl.ds(ebase, _AGG_K)], sidx_v)
        pltpu.sync_copy(dst_hbm.at[pl.ds(ebase, _AGG_K)], didx_v)
        for j in range(_AGG_K // 16):
            sl = pl.ds(j * 16, 16)
            sidx_v[sl] = sidx_v[sl] + off
        pltpu.async_copy(zcat_hbm.at[sidx_v], rows_v, sem).wait()
        pltpu.sync_copy(rows_v, acc_s.at[didx_v], add=True)
        return carry

    lax.fori_loop(0, _AGG_CHUNKS, chunk, 0)
    plsc.subcore_barrier()
    pltpu.sync_copy(acc_s.at[pl.ds(row0, _RPT)],
                    out_hbm.at[pl.ds(c * N + row0, _RPT)])


_BR = 400


def _zdis_body(p0_ref, p1_ref, x_ref, zc_ref, disb_ref):
    sdeg = p0_ref[...] + p1_ref[...] + 1.0
    dis = lax.rsqrt(sdeg)
    z = x_ref[...] * dis
    zc_ref[0, :, :] = z[:, :H]
    zc_ref[1, :, :] = z[:, H:]
    disb_ref[...] = jnp.broadcast_to(dis, (_BR, H))


def _weights_body(wg_ref, wl_ref, bg_ref, bl_ref, wc_ref, c_ref):
    wl = wl_ref[...]
    wc_ref[...] = jnp.dot(wg_ref[...], wl, preferred_element_type=jnp.float32)
    c_ref[...] = (jnp.dot(bg_ref[...], wl, preferred_element_type=jnp.float32)
                  + bl_ref[...])


def _out_body(t0_ref, t1_ref, disb_ref, wc_ref, c_ref, o_ref):
    db = disb_ref[...]
    m = jnp.concatenate([t0_ref[...] * db, t1_ref[...] * db], axis=1)
    o_ref[...] = (jnp.dot(m, wc_ref[...], preferred_element_type=jnp.float32)
                  + c_ref[...])


def kernel(x, edge_index, W_gcn, b_gcn, W_lin, b_lin):
    ei = edge_index.astype(jnp.int32)
    src = ei[0]
    dst = ei[1]

    degp = _deg_kernel(dst)
    p0 = degp[:N].reshape(N, 1)
    p1 = degp[N_PAD:N_PAD + N].reshape(N, 1)

    zc, disb = pl.pallas_call(
        _zdis_body,
        grid=(N // _BR,),
        in_specs=[
            pl.BlockSpec((_BR, 1), lambda i: (i, 0)),
            pl.BlockSpec((_BR, 1), lambda i: (i, 0)),
            pl.BlockSpec((_BR, D), lambda i: (i, 0)),
        ],
        out_specs=[
            pl.BlockSpec((2, _BR, H), lambda i: (0, i, 0)),
            pl.BlockSpec((_BR, H), lambda i: (i, 0)),
        ],
        out_shape=[
            jax.ShapeDtypeStruct((2, N, H), jnp.float32),
            jax.ShapeDtypeStruct((N, H), jnp.float32),
        ],
    )(p0, p1, x)

    wc, cvec = pl.pallas_call(
        _weights_body,
        out_shape=[
            jax.ShapeDtypeStruct((D, D), jnp.float32),
            jax.ShapeDtypeStruct((1, D), jnp.float32),
        ],
    )(W_gcn, W_lin, b_gcn.reshape(1, D), b_lin.reshape(1, D))

    t = _agg_kernel(zc.reshape(NC * N, H), src, dst)

    out = pl.pallas_call(
        _out_body,
        grid=(N // _BR,),
        in_specs=[
            pl.BlockSpec((_BR, H), lambda i: (i, 0)),
            pl.BlockSpec((_BR, H), lambda i: (i, 0)),
            pl.BlockSpec((_BR, H), lambda i: (i, 0)),
            pl.BlockSpec((D, D), lambda i: (0, 0)),
            pl.BlockSpec((1, D), lambda i: (0, 0)),
        ],
        out_specs=pl.BlockSpec((_BR, D), lambda i: (i, 0)),
        out_shape=jax.ShapeDtypeStruct((N, D), jnp.float32),
    )(t[:N], t[N:], disb, wc, cvec)

    return out


# speedup vs baseline: 16.8237x; 1.8278x over previous
import functools

import jax
import jax.numpy as jnp
from jax import lax
from jax.experimental import pallas as pl
from jax.experimental.pallas import tpu as pltpu
from jax.experimental.pallas import tpu_sc as plsc

N = 10000
E = 160000
D = 256
H = D // 2
NC = 2
NS = 16
N_PAD = 10240

_sc_mesh = plsc.VectorSubcoreMesh(core_axis_name="c", subcore_axis_name="s")

_DEG_K = 40
_DEG_EPT = E // (NC * NS)
_DEG_CHUNKS = _DEG_EPT // _DEG_K
_ZPT = N_PAD // NS
_NBUF = 4


@functools.partial(
    pl.kernel,
    out_type=jax.ShapeDtypeStruct((NC * N_PAD,), jnp.float32),
    mesh=_sc_mesh,
    scratch_types=[
        pltpu.VMEM((48,), jnp.float32),
        pltpu.VMEM((_ZPT,), jnp.float32),
        pltpu.VMEM((_DEG_CHUNKS, _DEG_K), jnp.int32),
        pltpu.VMEM_SHARED((N_PAD,), jnp.float32),
    ] + [pltpu.SemaphoreType.DMA] * _NBUF,
)
def _deg_kernel(dst3_hbm, out_hbm, ones_v, zeros_v, dbuf, acc_s, *ssem):
    c = lax.axis_index("c")
    s = lax.axis_index("s")
    w = c * NS + s
    for j in range(3):
        ones_v[pl.ds(j * 16, 16)] = jnp.ones((16,), jnp.float32)

    def zfill(i, carry):
        zeros_v[pl.ds(i * 16, 16)] = jnp.zeros((16,), jnp.float32)
        return carry

    lax.fori_loop(0, _ZPT // 16, zfill, 0)
    pltpu.sync_copy(zeros_v, acc_s.at[pl.ds(s * _ZPT, _ZPT)])
    pltpu.sync_copy(dst3_hbm.at[w], dbuf)
    plsc.subcore_barrier()

    ones40 = ones_v.at[pl.ds(0, _DEG_K)]

    def _scat(g, j):
        return pltpu.async_copy(ones40, acc_s.at[dbuf.at[g]], ssem[j], add=True)

    def _scat_wait(j):
        pltpu.make_async_copy(ones40, acc_s.at[dbuf.at[0]], ssem[j]).wait()

    def quad(k, carry):
        for j in range(_NBUF):
            @pl.when(k > 0)
            def _():
                _scat_wait(j)
            _scat(k * _NBUF + j, j)
        return carry

    lax.fori_loop(0, 31, quad, 0)
    _scat_wait(0)
    _scat(124, 0)
    for j in range(_NBUF):
        _scat_wait(j)
    plsc.subcore_barrier()
    pltpu.sync_copy(acc_s.at[pl.ds(s * _ZPT, _ZPT)],
                    out_hbm.at[pl.ds(c * N_PAD + s * _ZPT, _ZPT)])


_AGG_K = 80
_AGG_EPT = E // NS
_AGG_CHUNKS = _AGG_EPT // _AGG_K
_RPT = 640
_RSTRIDE = 624
_AGG_NBUF = 2


@functools.partial(
    pl.kernel,
    out_type=jax.ShapeDtypeStruct((NC * N, H), jnp.float32),
    mesh=_sc_mesh,
    scratch_types=[
        pltpu.VMEM((_AGG_EPT,), jnp.int32),
        pltpu.VMEM((_AGG_CHUNKS, _AGG_K), jnp.int32),
        pltpu.VMEM((_AGG_NBUF, _AGG_K, H), jnp.float32),
        pltpu.VMEM_SHARED((N, H), jnp.float32),
    ] + [pltpu.SemaphoreType.DMA] * (2 * _AGG_NBUF),
)
def _agg_kernel(zcat_hbm, src_hbm, dst3_hbm, out_hbm,
                sbuf, dbuf, rows_v, acc_s, *sems):
    gsem = sems[:_AGG_NBUF]
    ssem = sems[_AGG_NBUF:]
    c = lax.axis_index("c")
    s = lax.axis_index("s")
    row0 = s * _RSTRIDE
    pltpu.sync_copy(src_hbm.at[pl.ds(s * _AGG_EPT, _AGG_EPT)], sbuf)
    pltpu.sync_copy(dst3_hbm.at[s], dbuf)
    off = c * N

    def adj(i, carry):
        sl = pl.ds(i * 16, 16)
        sbuf[sl] = sbuf[sl] + off
        return carry

    lax.fori_loop(0, _AGG_EPT // 16, adj, 0)
    pltpu.sync_copy(zcat_hbm.at[pl.ds(c * N + row0, _RPT)],
                    acc_s.at[pl.ds(row0, _RPT)])
    plsc.subcore_barrier()

    def _gat(g, j):
        pltpu.async_copy(zcat_hbm.at[sbuf.at[pl.ds(g * _AGG_K, _AGG_K)]],
                         rows_v.at[j], gsem[j])

    def _gat_wait(j):
        pltpu.make_async_copy(zcat_hbm.at[sbuf.at[pl.ds(0, _AGG_K)]],
                              rows_v.at[j], gsem[j]).wait()

    def _scat(g, j):
        pltpu.async_copy(rows_v.at[j], acc_s.at[dbuf.at[g]], ssem[j], add=True)

    def _scat_wait(j):
        pltpu.make_async_copy(rows_v.at[j], acc_s.at[dbuf.at[0]], ssem[j]).wait()

    for j in range(_AGG_NBUF):
        _gat(j, j)

    def pair(k, carry):
        for j in range(_AGG_NBUF):
            _gat_wait(j)
            _scat(k * _AGG_NBUF + j, j)
        for j in range(_AGG_NBUF):
            g = (k + 1) * _AGG_NBUF + j
            @pl.when(g < _AGG_CHUNKS)
            def _():
                _scat_wait(j)
                _gat(g, j)
        return carry

    lax.fori_loop(0, 62, pair, 0)
    _gat_wait(0)
    _scat(124, 0)
    for j in range(_AGG_NBUF):
        _scat_wait(j)
    plsc.subcore_barrier()
    pltpu.sync_copy(acc_s.at[pl.ds(row0, _RPT)],
                    out_hbm.at[pl.ds(c * N + row0, _RPT)])


_BR = 400


def _zdis_body(p0_ref, p1_ref, x_ref, zc_ref, disb_ref):
    sdeg = p0_ref[...] + p1_ref[...] + 1.0
    dis = lax.rsqrt(sdeg)
    z = x_ref[...] * dis
    zc_ref[0, :, :] = z[:, :H]
    zc_ref[1, :, :] = z[:, H:]
    disb_ref[...] = jnp.broadcast_to(dis, (_BR, H))


def _weights_body(wg_ref, wl_ref, bg_ref, bl_ref, wc_ref, c_ref):
    wl = wl_ref[...]
    wc_ref[...] = jnp.dot(wg_ref[...], wl, preferred_element_type=jnp.float32)
    c_ref[...] = (jnp.dot(bg_ref[...], wl, preferred_element_type=jnp.float32)
                  + bl_ref[...])


def _out_body(t0_ref, t1_ref, disb_ref, wc_ref, c_ref, o_ref):
    db = disb_ref[...]
    m = jnp.concatenate([t0_ref[...] * db, t1_ref[...] * db], axis=1)
    o_ref[...] = (jnp.dot(m, wc_ref[...], preferred_element_type=jnp.float32)
                  + c_ref[...])


def kernel(x, edge_index, W_gcn, b_gcn, W_lin, b_lin):
    ei = edge_index.astype(jnp.int32)
    src = ei[0]
    dst = ei[1]

    degp = _deg_kernel(dst.reshape(NC * NS, _DEG_CHUNKS, _DEG_K))
    p0 = degp[:N].reshape(N, 1)
    p1 = degp[N_PAD:N_PAD + N].reshape(N, 1)

    zc, disb = pl.pallas_call(
        _zdis_body,
        grid=(N // _BR,),
        in_specs=[
            pl.BlockSpec((_BR, 1), lambda i: (i, 0)),
            pl.BlockSpec((_BR, 1), lambda i: (i, 0)),
            pl.BlockSpec((_BR, D), lambda i: (i, 0)),
        ],
        out_specs=[
            pl.BlockSpec((2, _BR, H), lambda i: (0, i, 0)),
            pl.BlockSpec((_BR, H), lambda i: (i, 0)),
        ],
        out_shape=[
            jax.ShapeDtypeStruct((2, N, H), jnp.float32),
            jax.ShapeDtypeStruct((N, H), jnp.float32),
        ],
    )(p0, p1, x)

    wc, cvec = pl.pallas_call(
        _weights_body,
        out_shape=[
            jax.ShapeDtypeStruct((D, D), jnp.float32),
            jax.ShapeDtypeStruct((1, D), jnp.float32),
        ],
    )(W_gcn, W_lin, b_gcn.reshape(1, D), b_lin.reshape(1, D))

    t = _agg_kernel(zc.reshape(NC * N, H), src,
                    dst.reshape(NS, _AGG_CHUNKS, _AGG_K))

    out = pl.pallas_call(
        _out_body,
        grid=(N // _BR,),
        in_specs=[
            pl.BlockSpec((_BR, H), lambda i: (i, 0)),
            pl.BlockSpec((_BR, H), lambda i: (i, 0)),
            pl.BlockSpec((_BR, H), lambda i: (i, 0)),
            pl.BlockSpec((D, D), lambda i: (0, 0)),
            pl.BlockSpec((1, D), lambda i: (0, 0)),
        ],
        out_specs=pl.BlockSpec((_BR, D), lambda i: (i, 0)),
        out_shape=jax.ShapeDtypeStruct((N, D), jnp.float32),
    )(t[:N], t[N:], disb, wc, cvec)

    return out


# speedup vs baseline: 20.5969x; 1.2243x over previous
import functools

import jax
import jax.numpy as jnp
from jax import lax
from jax.experimental import pallas as pl
from jax.experimental.pallas import tpu as pltpu
from jax.experimental.pallas import tpu_sc as plsc

N = 10000
E = 160000
D = 256
H = D // 2
NC = 2
NS = 16
N_PAD = 10240

_K = 80
_EPT = E // NS
_CHUNKS = _EPT // _K
_ZPT = N_PAD // NS
_RPT = 640
_RSTRIDE = 624

_sc_mesh = plsc.VectorSubcoreMesh(core_axis_name="c", subcore_axis_name="s")


_DEG_NBUF = 4
_DEG_SPLIT = 62


@functools.partial(
    pl.kernel,
    out_type=jax.ShapeDtypeStruct((NC * N_PAD,), jnp.float32),
    mesh=_sc_mesh,
    scratch_types=[
        pltpu.VMEM((_K,), jnp.float32),
        pltpu.VMEM((_ZPT,), jnp.float32),
        pltpu.VMEM((_DEG_NBUF, _K), jnp.int32),
        pltpu.VMEM_SHARED((N_PAD,), jnp.float32),
    ] + [pltpu.SemaphoreType.DMA] * (2 * _DEG_NBUF),
)
def _deg_kernel(dst_hbm, out_hbm, ones_v, zeros_v, didx, acc_s, *sems):
    dsem = sems[:_DEG_NBUF]
    ssem = sems[_DEG_NBUF:]
    c = lax.axis_index("c")
    s = lax.axis_index("s")
    for j in range(_K // 16):
        ones_v[pl.ds(j * 16, 16)] = jnp.ones((16,), jnp.float32)

    def zfill(i, carry):
        zeros_v[pl.ds(i * 16, 16)] = jnp.zeros((16,), jnp.float32)
        return carry

    lax.fori_loop(0, _ZPT // 16, zfill, 0)
    pltpu.sync_copy(zeros_v, acc_s.at[pl.ds(s * _ZPT, _ZPT)])
    plsc.subcore_barrier()

    ebase0 = s * _EPT

    def _idx(g, j):
        pltpu.async_copy(dst_hbm.at[pl.ds(ebase0 + g * _K, _K)],
                         didx.at[j], dsem[j])

    def _idx_wait(j):
        pltpu.make_async_copy(dst_hbm.at[pl.ds(0, _K)], didx.at[j],
                              dsem[j]).wait()

    def _scat(j):
        pltpu.async_copy(ones_v, acc_s.at[didx.at[j]], ssem[j], add=True)

    def _scat_wait(j):
        pltpu.make_async_copy(ones_v, acc_s.at[didx.at[0]], ssem[j]).wait()

    def _pipeline(lo, hi):
        n = hi - lo
        nb = min(_DEG_NBUF, n)
        for j in range(nb):
            _idx(lo + j, j)
        nq = (n + _DEG_NBUF - 1) // _DEG_NBUF

        def quad(k, carry):
            for j in range(_DEG_NBUF):
                g = k * _DEG_NBUF + j

                @pl.when(g < n)
                def _():
                    _idx_wait(j)
                    _scat(j)
                g2 = (k + 1) * _DEG_NBUF + j

                @pl.when(g2 < n)
                def _():
                    _scat_wait(j)
                    _idx(lo + g2, j)
            return carry

        lax.fori_loop(0, nq, quad, 0)
        for j in range(nb):
            _scat_wait(j)

    @pl.when(c == 0)
    def _():
        _pipeline(0, _DEG_SPLIT)

    @pl.when(c == 1)
    def _():
        _pipeline(_DEG_SPLIT, _CHUNKS)

    plsc.subcore_barrier()
    pltpu.sync_copy(acc_s.at[pl.ds(s * _ZPT, _ZPT)],
                    out_hbm.at[pl.ds(c * N_PAD + s * _ZPT, _ZPT)])


_AGG_NBUF = 3


@functools.partial(
    pl.kernel,
    out_type=jax.ShapeDtypeStruct((NC * N, H), jnp.float32),
    mesh=_sc_mesh,
    scratch_types=[
        pltpu.VMEM((_EPT,), jnp.int32),
        pltpu.VMEM((_AGG_NBUF, _K), jnp.int32),
        pltpu.VMEM((_AGG_NBUF, _K, H), jnp.float32),
        pltpu.VMEM_SHARED((N, H), jnp.float32),
    ] + [pltpu.SemaphoreType.DMA] * (3 * _AGG_NBUF),
)
def _agg_kernel(zcat_hbm, src_hbm, dst_hbm, out_hbm,
                sbuf, didx, rows_v, acc_s, *sems):
    dsem = sems[:_AGG_NBUF]
    gsem = sems[_AGG_NBUF:2 * _AGG_NBUF]
    ssem = sems[2 * _AGG_NBUF:]
    c = lax.axis_index("c")
    s = lax.axis_index("s")
    row0 = s * _RSTRIDE
    pltpu.sync_copy(src_hbm.at[pl.ds(s * _EPT, _EPT)], sbuf)
    off = c * N

    def adj(i, carry):
        sl = pl.ds(i * 16, 16)
        sbuf[sl] = sbuf[sl] + off
        return carry

    lax.fori_loop(0, _EPT // 16, adj, 0)
    pltpu.sync_copy(zcat_hbm.at[pl.ds(c * N + row0, _RPT)],
                    acc_s.at[pl.ds(row0, _RPT)])
    plsc.subcore_barrier()

    ebase0 = s * _EPT

    def _start(g, j):
        pltpu.async_copy(dst_hbm.at[pl.ds(ebase0 + g * _K, _K)],
                         didx.at[j], dsem[j])
        pltpu.async_copy(zcat_hbm.at[sbuf.at[pl.ds(g * _K, _K)]],
                         rows_v.at[j], gsem[j])

    def _start_wait(j):
        pltpu.make_async_copy(dst_hbm.at[pl.ds(0, _K)], didx.at[j],
                              dsem[j]).wait()
        pltpu.make_async_copy(zcat_hbm.at[sbuf.at[pl.ds(0, _K)]],
                              rows_v.at[j], gsem[j]).wait()

    def _scat(j):
        pltpu.async_copy(rows_v.at[j], acc_s.at[didx.at[j]], ssem[j], add=True)

    def _scat_wait(j):
        pltpu.make_async_copy(rows_v.at[j], acc_s.at[didx.at[0]],
                              ssem[j]).wait()

    for j in range(_AGG_NBUF):
        _start(j, j)
    ntri = (_CHUNKS + _AGG_NBUF - 1) // _AGG_NBUF

    def tri(k, carry):
        for j in range(_AGG_NBUF):
            g = k * _AGG_NBUF + j

            @pl.when(g < _CHUNKS)
            def _():
                _start_wait(j)
                _scat(j)
            g2 = (k + 1) * _AGG_NBUF + j

            @pl.when(g2 < _CHUNKS)
            def _():
                _scat_wait(j)
                _start(g2, j)
        return carry

    lax.fori_loop(0, ntri, tri, 0)
    for j in range(_AGG_NBUF):
        _scat_wait(j)
    plsc.subcore_barrier()
    pltpu.sync_copy(acc_s.at[pl.ds(row0, _RPT)],
                    out_hbm.at[pl.ds(c * N + row0, _RPT)])


_BR = 400


def _z_body(p0_ref, p1_ref, x_ref, z_ref):
    i = pl.program_id(0)
    sdeg = p0_ref[...] + p1_ref[...] + 1.0
    dis = lax.rsqrt(sdeg)
    z = x_ref[...] * dis
    z_ref[...] = jnp.where(i < N // _BR, z[:, :H], z[:, H:])


def _weights_body(wg_ref, wl_ref, bg_ref, bl_ref, wc_ref, c_ref):
    wl = wl_ref[...]
    wc_ref[...] = jnp.dot(wg_ref[...], wl, preferred_element_type=jnp.float32)
    c_ref[...] = (jnp.dot(bg_ref[...], wl, preferred_element_type=jnp.float32)
                  + bl_ref[...])


def _out_body(t0_ref, t1_ref, p0_ref, p1_ref, wc_ref, c_ref, o_ref):
    dis = lax.rsqrt(p0_ref[...] + p1_ref[...] + 1.0)
    m = jnp.concatenate([t0_ref[...], t1_ref[...]], axis=1) * dis
    o_ref[...] = (jnp.dot(m, wc_ref[...], preferred_element_type=jnp.float32)
                  + c_ref[...])


def kernel(x, edge_index, W_gcn, b_gcn, W_lin, b_lin):
    ei = edge_index.astype(jnp.int32)
    src = ei[0]
    dst = ei[1]

    degp = _deg_kernel(dst)
    p0 = degp[:N].reshape(N, 1)
    p1 = degp[N_PAD:N_PAD + N].reshape(N, 1)

    nb = N // _BR
    zcat = pl.pallas_call(
        _z_body,
        grid=(2 * nb,),
        in_specs=[
            pl.BlockSpec((_BR, 1), lambda i: (lax.rem(i, nb), 0)),
            pl.BlockSpec((_BR, 1), lambda i: (lax.rem(i, nb), 0)),
            pl.BlockSpec((_BR, D), lambda i: (lax.rem(i, nb), 0)),
        ],
        out_specs=pl.BlockSpec((_BR, H), lambda i: (i, 0)),
        out_shape=jax.ShapeDtypeStruct((NC * N, H), jnp.float32),
    )(p0, p1, x)

    wc, cvec = pl.pallas_call(
        _weights_body,
        out_shape=[
            jax.ShapeDtypeStruct((D, D), jnp.float32),
            jax.ShapeDtypeStruct((1, D), jnp.float32),
        ],
    )(W_gcn, W_lin, b_gcn.reshape(1, D), b_lin.reshape(1, D))

    t = _agg_kernel(zcat, src, dst)

    out = pl.pallas_call(
        _out_body,
        grid=(nb,),
        in_specs=[
            pl.BlockSpec((_BR, H), lambda i: (i, 0)),
            pl.BlockSpec((_BR, H), lambda i: (i + nb, 0)),
            pl.BlockSpec((_BR, 1), lambda i: (i, 0)),
            pl.BlockSpec((_BR, 1), lambda i: (i, 0)),
            pl.BlockSpec((D, D), lambda i: (0, 0)),
            pl.BlockSpec((1, D), lambda i: (0, 0)),
        ],
        out_specs=pl.BlockSpec((_BR, D), lambda i: (i, 0)),
        out_shape=jax.ShapeDtypeStruct((N, D), jnp.float32),
    )(t, t, p0, p1, wc, cvec)

    return out


# speedup vs baseline: 23.2011x; 1.1264x over previous
import functools

import jax
import jax.numpy as jnp
from jax import lax
from jax.experimental import pallas as pl
from jax.experimental.pallas import tpu as pltpu
from jax.experimental.pallas import tpu_sc as plsc

N = 10000
E = 160000
D = 256
H = D // 2
NC = 2
NS = 16
N_PAD = 10240

_K = 80
_EPT = E // NS
_CHUNKS = _EPT // _K
_ZPT = N_PAD // NS
_RPT = 640
_RSTRIDE = 624
_RQ = _RPT // _K
_NBUF = 3

_sc_mesh = plsc.VectorSubcoreMesh(core_axis_name="c", subcore_axis_name="s")


def _newton_rsqrt16(d):
    i = plsc.bitcast(d, jnp.int32)
    i = 0x5F3759DF - lax.shift_right_arithmetic(i, 1)
    y = plsc.bitcast(i, jnp.float32)
    for _ in range(3):
        y = y * (1.5 - 0.5 * d * y * y)
    return y


@functools.partial(
    pl.kernel,
    out_type=[
        jax.ShapeDtypeStruct((NC * N, H), jnp.float32),
        jax.ShapeDtypeStruct((NC * N, H), jnp.float32),
    ],
    mesh=_sc_mesh,
    scratch_types=[
        pltpu.VMEM((_K,), jnp.float32),
        pltpu.VMEM((_ZPT,), jnp.float32),
        pltpu.VMEM((_RPT + 16,), jnp.float32),
        pltpu.VMEM((_EPT,), jnp.int32),
        pltpu.VMEM((_NBUF, _K), jnp.int32),
        pltpu.VMEM((_NBUF, _K, H), jnp.float32),
        pltpu.VMEM_SHARED((N_PAD,), jnp.float32),
        pltpu.VMEM_SHARED((N, H), jnp.float32),
    ] + [pltpu.SemaphoreType.DMA] * (3 * _NBUF),
)
def _gcn_sc_kernel(x_hbm, src_hbm, dst_hbm, t_hbm, z_hbm,
                   ones_v, zeros_v, tb, sbuf, didx, rows_v, deg_s, acc_s,
                   *sems):
    dsem = sems[:_NBUF]
    gsem = sems[_NBUF:2 * _NBUF]
    ssem = sems[2 * _NBUF:]
    c = lax.axis_index("c")
    s = lax.axis_index("s")
    row0 = s * _RSTRIDE
    ebase0 = s * _EPT

    for j in range(_K // 16):
        ones_v[pl.ds(j * 16, 16)] = jnp.ones((16,), jnp.float32)

    def zfill(i, carry):
        zeros_v[pl.ds(i * 16, 16)] = jnp.zeros((16,), jnp.float32)
        return carry

    lax.fori_loop(0, _ZPT // 16, zfill, 0)
    pltpu.sync_copy(zeros_v, deg_s.at[pl.ds(s * _ZPT, _ZPT)])
    pltpu.sync_copy(src_hbm.at[pl.ds(ebase0, _EPT)], sbuf)
    off = c * N

    def adj(i, carry):
        sl = pl.ds(i * 16, 16)
        sbuf[sl] = sbuf[sl] + off
        return carry

    lax.fori_loop(0, _EPT // 16, adj, 0)
    plsc.subcore_barrier()

    def _didx(g, j):
        pltpu.async_copy(dst_hbm.at[pl.ds(ebase0 + g * _K, _K)],
                         didx.at[j], dsem[j])

    def _didx_wait(j):
        pltpu.make_async_copy(dst_hbm.at[pl.ds(0, _K)], didx.at[j],
                              dsem[j]).wait()

    def _hist(j):
        pltpu.async_copy(ones_v, deg_s.at[didx.at[j]], ssem[j], add=True)

    def _hist_wait(j):
        pltpu.make_async_copy(ones_v, deg_s.at[didx.at[0]], ssem[j]).wait()

    for j in range(_NBUF):
        _didx(j, j)
    nh = (_CHUNKS + _NBUF - 1) // _NBUF

    def hloop(k, carry):
        for j in range(_NBUF):
            g = k * _NBUF + j

            @pl.when(g < _CHUNKS)
            def _():
                _didx_wait(j)
                _hist(j)
            g2 = (k + 1) * _NBUF + j

            @pl.when(g2 < _CHUNKS)
            def _():
                _hist_wait(j)
                _didx(g2, j)
        return carry

    lax.fori_loop(0, nh, hloop, 0)
    for j in range(_NBUF):
        _hist_wait(j)
    plsc.subcore_barrier()

    pltpu.sync_copy(deg_s.at[pl.ds(row0, _RPT)], tb.at[pl.ds(0, _RPT)])

    def newt(i, carry):
        sl = pl.ds(i * 16, 16)
        d = tb[sl] + 1.0
        iv = lax.bitcast_convert_type(d, jnp.int32)
        iv = 0x5F3759DF - lax.shift_right_arithmetic(iv, 1)
        y = lax.bitcast_convert_type(iv, jnp.float32)
        for _ in range(3):
            y = y * (1.5 - 0.5 * d * y * y)
        tb[sl] = y
        return carry

    lax.fori_loop(0, _RPT // 16, newt, 0)

    def _scale_rows(j, qbase):
        lane0 = jnp.zeros((16,), jnp.int32)

        lane0 = jnp.zeros((16,), jnp.int32)

        def srow(r, carry):
            dvec = tb[pl.ds(qbase + r, 16)]
            dv = dvec.at[lane0].get(mode="promise_in_bounds")
            for w in range(H // 16):
                sl = pl.ds(w * 16, 16)
                rows_v[j, r, sl] = rows_v[j, r, sl] * dv
            return carry

        lax.fori_loop(0, _K, srow, 0)

    def _zphase(colref):
        for q in range(_RQ):
            rq = row0 + q * _K
            pltpu.sync_copy(colref.at[pl.ds(rq, _K)], rows_v.at[0])
            _scale_rows(0, q * _K)
            pltpu.sync_copy(rows_v.at[0], z_hbm.at[pl.ds(c * N + rq, _K)])
            pltpu.sync_copy(rows_v.at[0], acc_s.at[pl.ds(rq, _K)])

    @pl.when(c == 0)
    def _():
        _zphase(x_hbm.at[:, pl.ds(0, H)])

    @pl.when(c == 1)
    def _():
        _zphase(x_hbm.at[:, pl.ds(H, H)])

    plsc.subcore_barrier()

    def _start(g, j):
        pltpu.async_copy(dst_hbm.at[pl.ds(ebase0 + g * _K, _K)],
                         didx.at[j], dsem[j])
        pltpu.async_copy(z_hbm.at[sbuf.at[pl.ds(g * _K, _K)]],
                         rows_v.at[j], gsem[j])

    def _start_wait(j):
        pltpu.make_async_copy(dst_hbm.at[pl.ds(0, _K)], didx.at[j],
                              dsem[j]).wait()
        pltpu.make_async_copy(z_hbm.at[sbuf.at[pl.ds(0, _K)]],
                              rows_v.at[j], gsem[j]).wait()

    def _scat(j):
        pltpu.async_copy(rows_v.at[j], acc_s.at[didx.at[j]], ssem[j], add=True)

    def _scat_wait(j):
        pltpu.make_async_copy(rows_v.at[j], acc_s.at[didx.at[0]],
                              ssem[j]).wait()

    for j in range(_NBUF):
        _start(j, j)
    ntri = (_CHUNKS + _NBUF - 1) // _NBUF

    def tri(k, carry):
        for j in range(_NBUF):
            g = k * _NBUF + j

            @pl.when(g < _CHUNKS)
            def _():
                _start_wait(j)
                _scat(j)
            g2 = (k + 1) * _NBUF + j

            @pl.when(g2 < _CHUNKS)
            def _():
                _scat_wait(j)
                _start(g2, j)
        return carry

    lax.fori_loop(0, ntri, tri, 0)
    for j in range(_NBUF):
        _scat_wait(j)
    plsc.subcore_barrier()

    for q in range(_RQ):
        rq = row0 + q * _K
        pltpu.sync_copy(acc_s.at[pl.ds(rq, _K)], rows_v.at[0])
        _scale_rows(0, q * _K)
        pltpu.sync_copy(rows_v.at[0], t_hbm.at[pl.ds(c * N + rq, _K)])


_BR = 1000


def _weights_body(wg_ref, wl_ref, bg_ref, bl_ref, wc_ref, c_ref):
    wl = wl_ref[...]
    wc_ref[...] = jnp.dot(wg_ref[...], wl, preferred_element_type=jnp.float32)
    c_ref[...] = (jnp.dot(bg_ref[...], wl, preferred_element_type=jnp.float32)
                  + bl_ref[...])


def _out_body(t0_ref, t1_ref, wc_ref, c_ref, o_ref):
    m = jnp.concatenate([t0_ref[...], t1_ref[...]], axis=1)
    o_ref[...] = (jnp.dot(m, wc_ref[...], preferred_element_type=jnp.float32)
                  + c_ref[...])


def kernel(x, edge_index, W_gcn, b_gcn, W_lin, b_lin):
    ei = edge_index.astype(jnp.int32)
    src = ei[0]
    dst = ei[1]

    t, _ = _gcn_sc_kernel(x, src, dst)

    wc, cvec = pl.pallas_call(
        _weights_body,
        out_shape=[
            jax.ShapeDtypeStruct((D, D), jnp.float32),
            jax.ShapeDtypeStruct((1, D), jnp.float32),
        ],
    )(W_gcn, W_lin, b_gcn.reshape(1, D), b_lin.reshape(1, D))

    nb = N // _BR
    out = pl.pallas_call(
        _out_body,
        grid=(nb,),
        in_specs=[
            pl.BlockSpec((_BR, H), lambda i: (i, 0)),
            pl.BlockSpec((_BR, H), lambda i: (i + nb, 0)),
            pl.BlockSpec((D, D), lambda i: (0, 0)),
            pl.BlockSpec((1, D), lambda i: (0, 0)),
        ],
        out_specs=pl.BlockSpec((_BR, D), lambda i: (i, 0)),
        out_shape=jax.ShapeDtypeStruct((N, D), jnp.float32),
    )(t, t, wc, cvec)

    return out


# speedup vs baseline: 24.8391x; 1.0706x over previous
import functools

import jax
import jax.numpy as jnp
from jax import lax
from jax.experimental import pallas as pl
from jax.experimental.pallas import tpu as pltpu
from jax.experimental.pallas import tpu_sc as plsc

N = 10000
E = 160000
D = 256
H = D // 2
NC = 2
NS = 16
N_PAD = 10240

_K = 80
_EPT = E // NS
_CHUNKS = _EPT // _K
_ZPT = N_PAD // NS
_RPT = 640
_RSTRIDE = 624
_RQ = _RPT // _K
_NBUF = 3

_sc_mesh = plsc.VectorSubcoreMesh(core_axis_name="c", subcore_axis_name="s")


def _newton_rsqrt16(d):
    i = plsc.bitcast(d, jnp.int32)
    i = 0x5F3759DF - lax.shift_right_arithmetic(i, 1)
    y = plsc.bitcast(i, jnp.float32)
    for _ in range(3):
        y = y * (1.5 - 0.5 * d * y * y)
    return y


@functools.partial(
    pl.kernel,
    out_type=[
        jax.ShapeDtypeStruct((NC * N, H), jnp.float32),
        jax.ShapeDtypeStruct((NC * N, H), jnp.float32),
    ],
    mesh=_sc_mesh,
    scratch_types=[
        pltpu.VMEM((_K,), jnp.float32),
        pltpu.VMEM((_ZPT,), jnp.float32),
        pltpu.VMEM((_RPT + 16,), jnp.float32),
        pltpu.VMEM((_EPT,), jnp.int32),
        pltpu.VMEM((_NBUF, _K), jnp.int32),
        pltpu.VMEM((_NBUF, _K, H), jnp.float32),
        pltpu.VMEM_SHARED((N_PAD,), jnp.float32),
        pltpu.VMEM_SHARED((N, H), jnp.float32),
    ] + [pltpu.SemaphoreType.DMA] * (3 * _NBUF),
)
def _gcn_sc_kernel(x_hbm, ei_hbm, t_hbm, z_hbm,
                   ones_v, zeros_v, tb, sbuf, didx, rows_v, deg_s, acc_s,
                   *sems):
    dsem = sems[:_NBUF]
    gsem = sems[_NBUF:2 * _NBUF]
    ssem = sems[2 * _NBUF:]
    c = lax.axis_index("c")
    s = lax.axis_index("s")
    row0 = s * _RSTRIDE
    ebase0 = s * _EPT

    for j in range(_K // 16):
        ones_v[pl.ds(j * 16, 16)] = jnp.ones((16,), jnp.float32)

    def zfill(i, carry):
        zeros_v[pl.ds(i * 16, 16)] = jnp.zeros((16,), jnp.float32)
        return carry

    lax.fori_loop(0, _ZPT // 16, zfill, 0)
    pltpu.sync_copy(zeros_v, deg_s.at[pl.ds(s * _ZPT, _ZPT)])
    pltpu.sync_copy(ei_hbm.at[pl.ds(ebase0, _EPT)], sbuf)
    off = c * N

    def adj(i, carry):
        sl = pl.ds(i * 16, 16)
        sbuf[sl] = sbuf[sl] + off
        return carry

    lax.fori_loop(0, _EPT // 16, adj, 0)
    plsc.subcore_barrier()

    def _didx(g, j):
        pltpu.async_copy(ei_hbm.at[pl.ds(E + ebase0 + g * _K, _K)],
                         didx.at[j], dsem[j])

    def _didx_wait(j):
        pltpu.make_async_copy(ei_hbm.at[pl.ds(0, _K)], didx.at[j],
                              dsem[j]).wait()

    def _hist(j):
        pltpu.async_copy(ones_v, deg_s.at[didx.at[j]], ssem[j], add=True)

    def _hist_wait(j):
        pltpu.make_async_copy(ones_v, deg_s.at[didx.at[0]], ssem[j]).wait()

    for j in range(_NBUF):
        _didx(j, j)
    nh = (_CHUNKS + _NBUF - 1) // _NBUF

    def hloop(k, carry):
        for j in range(_NBUF):
            g = k * _NBUF + j

            @pl.when(g < _CHUNKS)
            def _():
                _didx_wait(j)
                _hist(j)
            g2 = (k + 1) * _NBUF + j

            @pl.when(g2 < _CHUNKS)
            def _():
                _hist_wait(j)
                _didx(g2, j)
        return carry

    lax.fori_loop(0, nh, hloop, 0)
    for j in range(_NBUF):
        _hist_wait(j)
    plsc.subcore_barrier()

    pltpu.sync_copy(deg_s.at[pl.ds(row0, _RPT)], tb.at[pl.ds(0, _RPT)])

    def newt(i, carry):
        sl = pl.ds(i * 16, 16)
        d = tb[sl] + 1.0
        iv = lax.bitcast_convert_type(d, jnp.int32)
        iv = 0x5F3759DF - lax.shift_right_arithmetic(iv, 1)
        y = lax.bitcast_convert_type(iv, jnp.float32)
        for _ in range(3):
            y = y * (1.5 - 0.5 * d * y * y)
        tb[sl] = y
        return carry

    lax.fori_loop(0, _RPT // 16, newt, 0)

    def _scale_rows(j, qbase):
        lane0 = jnp.zeros((16,), jnp.int32)

        lane0 = jnp.zeros((16,), jnp.int32)

        def srow(r, carry):
            dvec = tb[pl.ds(qbase + r, 16)]
            dv = dvec.at[lane0].get(mode="promise_in_bounds")
            for w in range(H // 16):
                sl = pl.ds(w * 16, 16)
                rows_v[j, r, sl] = rows_v[j, r, sl] * dv
            return carry

        lax.fori_loop(0, _K, srow, 0)

    def _zphase(colref):
        def _xload(q, j):
            pltpu.async_copy(colref.at[pl.ds(row0 + q * _K, _K)],
                             rows_v.at[j], gsem[j])

        def _xload_wait(j):
            pltpu.make_async_copy(colref.at[pl.ds(row0, _K)],
                                  rows_v.at[j], gsem[j]).wait()

        def _zw_wait(j):
            pltpu.make_async_copy(rows_v.at[j], z_hbm.at[pl.ds(0, _K)],
                                  ssem[j]).wait()
            pltpu.make_async_copy(rows_v.at[j], acc_s.at[pl.ds(0, _K)],
                                  dsem[j]).wait()

        _xload(0, 0)
        for q in range(_RQ):
            j = q & 1
            rq = row0 + q * _K
            _xload_wait(j)
            _scale_rows(j, q * _K)
            pltpu.async_copy(rows_v.at[j], z_hbm.at[pl.ds(c * N + rq, _K)],
                             ssem[j])
            pltpu.async_copy(rows_v.at[j], acc_s.at[pl.ds(rq, _K)], dsem[j])
            if q + 1 < _RQ:
                jn = (q + 1) & 1
                if q + 1 >= 2:
                    _zw_wait(jn)
                _xload(q + 1, jn)
        _zw_wait(_RQ & 1)
        _zw_wait((_RQ - 1) & 1)

    @pl.when(c == 0)
    def _():
        _zphase(x_hbm.at[:, pl.ds(0, H)])

    @pl.when(c == 1)
    def _():
        _zphase(x_hbm.at[:, pl.ds(H, H)])

    plsc.subcore_barrier()

    def _start(g, j):
        pltpu.async_copy(ei_hbm.at[pl.ds(E + ebase0 + g * _K, _K)],
                         didx.at[j], dsem[j])
        pltpu.async_copy(z_hbm.at[sbuf.at[pl.ds(g * _K, _K)]],
                         rows_v.at[j], gsem[j])

    def _start_wait(j):
        pltpu.make_async_copy(ei_hbm.at[pl.ds(0, _K)], didx.at[j],
                              dsem[j]).wait()
        pltpu.make_async_copy(z_hbm.at[sbuf.at[pl.ds(0, _K)]],
                              rows_v.at[j], gsem[j]).wait()

    def _scat(j):
        pltpu.async_copy(rows_v.at[j], acc_s.at[didx.at[j]], ssem[j], add=True)

    def _scat_wait(j):
        pltpu.make_async_copy(rows_v.at[j], acc_s.at[didx.at[0]],
                              ssem[j]).wait()

    for j in range(_NBUF):
        _start(j, j)
    ntri = (_CHUNKS + _NBUF - 1) // _NBUF

    def tri(k, carry):
        for j in range(_NBUF):
            g = k * _NBUF + j

            @pl.when(g < _CHUNKS)
            def _():
                _start_wait(j)
                _scat(j)
            g2 = (k + 1) * _NBUF + j

            @pl.when(g2 < _CHUNKS)
            def _():
                _scat_wait(j)
                _start(g2, j)
        return carry

    lax.fori_loop(0, ntri, tri, 0)
    for j in range(_NBUF):
        _scat_wait(j)
    plsc.subcore_barrier()

    def _aload(q, j):
        pltpu.async_copy(acc_s.at[pl.ds(row0 + q * _K, _K)],
                         rows_v.at[j], gsem[j])

    def _aload_wait(j):
        pltpu.make_async_copy(acc_s.at[pl.ds(row0, _K)], rows_v.at[j],
                              gsem[j]).wait()

    def _tw_wait(j):
        pltpu.make_async_copy(rows_v.at[j], t_hbm.at[pl.ds(0, _K)],
                              ssem[j]).wait()

    _aload(0, 0)
    for q in range(_RQ):
        j = q & 1
        rq = row0 + q * _K
        _aload_wait(j)
        _scale_rows(j, q * _K)
        pltpu.async_copy(rows_v.at[j], t_hbm.at[pl.ds(c * N + rq, _K)],
                         ssem[j])
        if q + 1 < _RQ:
            jn = (q + 1) & 1
            if q + 1 >= 2:
                _tw_wait(jn)
            _aload(q + 1, jn)
    _tw_wait(_RQ & 1)
    _tw_wait((_RQ - 1) & 1)


_BR = 1000


def _weights_body(wg_ref, wl_ref, bg_ref, bl_ref, wc_ref, c_ref):
    wl = wl_ref[...]
    wc_ref[...] = jnp.dot(wg_ref[...], wl, preferred_element_type=jnp.float32)
    c_ref[...] = (jnp.dot(bg_ref[...], wl, preferred_element_type=jnp.float32)
                  + bl_ref[...])


def _out_body(t0_ref, t1_ref, wc_ref, c_ref, o_ref):
    m = jnp.concatenate([t0_ref[...], t1_ref[...]], axis=1)
    o_ref[...] = (jnp.dot(m, wc_ref[...], preferred_element_type=jnp.float32)
                  + c_ref[...])


def kernel(x, edge_index, W_gcn, b_gcn, W_lin, b_lin):
    ei = edge_index.astype(jnp.int32)

    t, _ = _gcn_sc_kernel(x, ei.reshape(-1))

    wc, cvec = pl.pallas_call(
        _weights_body,
        out_shape=[
            jax.ShapeDtypeStruct((D, D), jnp.float32),
            jax.ShapeDtypeStruct((1, D), jnp.float32),
        ],
    )(W_gcn, W_lin, b_gcn.reshape(1, D), b_lin.reshape(1, D))

    nb = N // _BR
    out = pl.pallas_call(
        _out_body,
        grid=(nb,),
        in_specs=[
            pl.BlockSpec((_BR, H), lambda i: (i, 0)),
            pl.BlockSpec((_BR, H), lambda i: (i + nb, 0)),
            pl.BlockSpec((D, D), lambda i: (0, 0)),
            pl.BlockSpec((1, D), lambda i: (0, 0)),
        ],
        out_specs=pl.BlockSpec((_BR, D), lambda i: (i, 0)),
        out_shape=jax.ShapeDtypeStruct((N, D), jnp.float32),
    )(t, t, wc, cvec)

    return out


# speedup vs baseline: 25.1645x; 1.0131x over previous
import functools

import jax
import jax.numpy as jnp
from jax import lax
from jax.experimental import pallas as pl
from jax.experimental.pallas import tpu as pltpu
from jax.experimental.pallas import tpu_sc as plsc

N = 10000
E = 160000
D = 256
H = D // 2
NC = 2
NS = 16
N_PAD = 10240

_K = 80
_EPT = E // NS
_CHUNKS = _EPT // _K
_ZPT = N_PAD // NS
_RPT = 640
_RSTRIDE = 624
_RQ = _RPT // _K
_NBUF = 3

_sc_mesh = plsc.VectorSubcoreMesh(core_axis_name="c", subcore_axis_name="s")


def _newton_rsqrt16(d):
    i = plsc.bitcast(d, jnp.int32)
    i = 0x5F3759DF - lax.shift_right_arithmetic(i, 1)
    y = plsc.bitcast(i, jnp.float32)
    for _ in range(3):
        y = y * (1.5 - 0.5 * d * y * y)
    return y


@functools.partial(
    pl.kernel,
    out_type=[
        jax.ShapeDtypeStruct((NC * N, H), jnp.float32),
        jax.ShapeDtypeStruct((NC * N, H), jnp.float32),
    ],
    mesh=_sc_mesh,
    scratch_types=[
        pltpu.VMEM((_K,), jnp.float32),
        pltpu.VMEM((_ZPT,), jnp.float32),
        pltpu.VMEM((_RPT + 16,), jnp.float32),
        pltpu.VMEM((_EPT,), jnp.int32),
        pltpu.VMEM((_NBUF, _K), jnp.int32),
        pltpu.VMEM((_NBUF, _K, H), jnp.float32),
        pltpu.VMEM_SHARED((N_PAD,), jnp.float32),
        pltpu.VMEM_SHARED((N, H), jnp.float32),
    ] + [pltpu.SemaphoreType.DMA] * (3 * _NBUF),
)
def _gcn_sc_kernel(x_hbm, ei_hbm, t_hbm, z_hbm,
                   ones_v, zeros_v, tb, sbuf, didx, rows_v, deg_s, acc_s,
                   *sems):
    dsem = sems[:_NBUF]
    gsem = sems[_NBUF:2 * _NBUF]
    ssem = sems[2 * _NBUF:]
    c = lax.axis_index("c")
    s = lax.axis_index("s")
    row0 = s * _RSTRIDE
    ebase0 = s * _EPT

    for j in range(_K // 16):
        ones_v[pl.ds(j * 16, 16)] = jnp.ones((16,), jnp.float32)

    def zfill(i, carry):
        zeros_v[pl.ds(i * 16, 16)] = jnp.zeros((16,), jnp.float32)
        return carry

    lax.fori_loop(0, _ZPT // 16, zfill, 0)
    pltpu.sync_copy(zeros_v, deg_s.at[pl.ds(s * _ZPT, _ZPT)])
    pltpu.sync_copy(ei_hbm.at[pl.ds(ebase0, _EPT)], sbuf)
    off = c * N

    def adj(i, carry):
        sl = pl.ds(i * 16, 16)
        sbuf[sl] = sbuf[sl] + off
        return carry

    lax.fori_loop(0, _EPT // 16, adj, 0)
    plsc.subcore_barrier()

    def _didx(g, j):
        pltpu.async_copy(ei_hbm.at[pl.ds(E + ebase0 + g * _K, _K)],
                         didx.at[j], dsem[j])

    def _didx_wait(j):
        pltpu.make_async_copy(ei_hbm.at[pl.ds(0, _K)], didx.at[j],
                              dsem[j]).wait()

    def _hist(j):
        pltpu.async_copy(ones_v, deg_s.at[didx.at[j]], ssem[j], add=True)

    def _hist_wait(j):
        pltpu.make_async_copy(ones_v, deg_s.at[didx.at[0]], ssem[j]).wait()

    for j in range(_NBUF):
        _didx(j, j)
    nh = (_CHUNKS + _NBUF - 1) // _NBUF

    def hloop(k, carry):
        for j in range(_NBUF):
            g = k * _NBUF + j

            @pl.when(g < _CHUNKS)
            def _():
                _didx_wait(j)
                _hist(j)
            g2 = (k + 1) * _NBUF + j

            @pl.when(g2 < _CHUNKS)
            def _():
                _hist_wait(j)
                _didx(g2, j)
        return carry

    lax.fori_loop(0, nh, hloop, 0)
    for j in range(_NBUF):
        _hist_wait(j)
    plsc.subcore_barrier()

    pltpu.sync_copy(deg_s.at[pl.ds(row0, _RPT)], tb.at[pl.ds(0, _RPT)])

    def newt(i, carry):
        sl = pl.ds(i * 16, 16)
        d = tb[sl] + 1.0
        iv = lax.bitcast_convert_type(d, jnp.int32)
        iv = 0x5F3759DF - lax.shift_right_arithmetic(iv, 1)
        y = lax.bitcast_convert_type(iv, jnp.float32)
        for _ in range(3):
            y = y * (1.5 - 0.5 * d * y * y)
        tb[sl] = y
        return carry

    lax.fori_loop(0, _RPT // 16, newt, 0)

    def _scale_rows(j, qbase):
        lane0 = jnp.zeros((16,), jnp.int32)

        lane0 = jnp.zeros((16,), jnp.int32)

        def srow(r, carry):
            dvec = tb[pl.ds(qbase + r, 16)]
            dv = dvec.at[lane0].get(mode="promise_in_bounds")
            for w in range(H // 16):
                sl = pl.ds(w * 16, 16)
                rows_v[j, r, sl] = rows_v[j, r, sl] * dv
            return carry

        lax.fori_loop(0, _K, srow, 0)

    def _zphase(colref):
        def _xload(q, j):
            pltpu.async_copy(colref.at[pl.ds(row0 + q * _K, _K)],
                             rows_v.at[j], gsem[j])

        def _xload_wait(j):
            pltpu.make_async_copy(colref.at[pl.ds(row0, _K)],
                                  rows_v.at[j], gsem[j]).wait()

        def _zw_wait(j):
            pltpu.make_async_copy(rows_v.at[j], z_hbm.at[pl.ds(0, _K)],
                                  ssem[j]).wait()
            pltpu.make_async_copy(rows_v.at[j], acc_s.at[pl.ds(0, _K)],
                                  dsem[j]).wait()

        _xload(0, 0)
        for q in range(_RQ):
            j = q & 1
            rq = row0 + q * _K
            _xload_wait(j)
            _scale_rows(j, q * _K)
            pltpu.async_copy(rows_v.at[j], z_hbm.at[pl.ds(c * N + rq, _K)],
                             ssem[j])
            pltpu.async_copy(rows_v.at[j], acc_s.at[pl.ds(rq, _K)], dsem[j])
            if q + 1 < _RQ:
                jn = (q + 1) & 1
                if q + 1 >= 2:
                    _zw_wait(jn)
                _xload(q + 1, jn)
        _zw_wait(_RQ & 1)
        _zw_wait((_RQ - 1) & 1)

    @pl.when(c == 0)
    def _():
        _zphase(x_hbm.at[:, pl.ds(0, H)])

    @pl.when(c == 1)
    def _():
        _zphase(x_hbm.at[:, pl.ds(H, H)])

    plsc.subcore_barrier()

    def _start(g, j):
        pltpu.async_copy(ei_hbm.at[pl.ds(E + ebase0 + g * _K, _K)],
                         didx.at[j], dsem[j])
        pltpu.async_copy(z_hbm.at[sbuf.at[pl.ds(g * _K, _K)]],
                         rows_v.at[j], gsem[j])

    def _start_wait(j):
        pltpu.make_async_copy(ei_hbm.at[pl.ds(0, _K)], didx.at[j],
                              dsem[j]).wait()
        pltpu.make_async_copy(z_hbm.at[sbuf.at[pl.ds(0, _K)]],
                              rows_v.at[j], gsem[j]).wait()

    def _scat(j):
        pltpu.async_copy(rows_v.at[j], acc_s.at[didx.at[j]], ssem[j], add=True)

    def _scat_wait(j):
        pltpu.make_async_copy(rows_v.at[j], acc_s.at[didx.at[0]],
                              ssem[j]).wait()

    for j in range(_NBUF):
        _start(j, j)
    ntri = (_CHUNKS + _NBUF - 1) // _NBUF

    def tri(k, carry):
        for j in range(_NBUF):
            g = k * _NBUF + j

            @pl.when(g < _CHUNKS)
            def _():
                _start_wait(j)
                _scat(j)
            g2 = (k + 1) * _NBUF + j

            @pl.when(g2 < _CHUNKS)
            def _():
                _scat_wait(j)
                _start(g2, j)
        return carry

    lax.fori_loop(0, ntri, tri, 0)
    for j in range(_NBUF):
        _scat_wait(j)
    plsc.subcore_barrier()

    def _aload(q, j):
        pltpu.async_copy(acc_s.at[pl.ds(row0 + q * _K, _K)],
                         rows_v.at[j], gsem[j])

    def _aload_wait(j):
        pltpu.make_async_copy(acc_s.at[pl.ds(row0, _K)], rows_v.at[j],
                              gsem[j]).wait()

    def _tw_wait(j):
        pltpu.make_async_copy(rows_v.at[j], t_hbm.at[pl.ds(0, _K)],
                              ssem[j]).wait()

    _aload(0, 0)
    for q in range(_RQ):
        j = q & 1
        rq = row0 + q * _K
        _aload_wait(j)
        _scale_rows(j, q * _K)
        pltpu.async_copy(rows_v.at[j], t_hbm.at[pl.ds(c * N + rq, _K)],
                         ssem[j])
        if q + 1 < _RQ:
            jn = (q + 1) & 1
            if q + 1 >= 2:
                _tw_wait(jn)
            _aload(q + 1, jn)
    _tw_wait(_RQ & 1)
    _tw_wait((_RQ - 1) & 1)


_BR = 2000


def _weights_body(wg_ref, wl_ref, bg_ref, bl_ref, wc_ref, c_ref):
    wl = wl_ref[...]
    wc_ref[...] = jnp.dot(wg_ref[...], wl, preferred_element_type=jnp.float32)
    c_ref[...] = (jnp.dot(bg_ref[...], wl, preferred_element_type=jnp.float32)
                  + bl_ref[...])


def _out_body(t0_ref, t1_ref, wc_ref, c_ref, o_ref):
    m = jnp.concatenate([t0_ref[...], t1_ref[...]], axis=1)
    o_ref[...] = (jnp.dot(m, wc_ref[...], preferred_element_type=jnp.float32)
                  + c_ref[...])


def kernel(x, edge_index, W_gcn, b_gcn, W_lin, b_lin):
    ei = edge_index.astype(jnp.int32)

    t, _ = _gcn_sc_kernel(x, ei.reshape(-1))

    wc, cvec = pl.pallas_call(
        _weights_body,
        out_shape=[
            jax.ShapeDtypeStruct((D, D), jnp.float32),
            jax.ShapeDtypeStruct((1, D), jnp.float32),
        ],
    )(W_gcn, W_lin, b_gcn.reshape(1, D), b_lin.reshape(1, D))

    nb = N // _BR
    out = pl.pallas_call(
        _out_body,
        grid=(nb,),
        in_specs=[
            pl.BlockSpec((_BR, H), lambda i: (i, 0)),
            pl.BlockSpec((_BR, H), lambda i: (i + nb, 0)),
            pl.BlockSpec((D, D), lambda i: (0, 0)),
            pl.BlockSpec((1, D), lambda i: (0, 0)),
        ],
        out_specs=pl.BlockSpec((_BR, D), lambda i: (i, 0)),
        out_shape=jax.ShapeDtypeStruct((N, D), jnp.float32),
    )(t, t, wc, cvec)

    return out


# speedup vs baseline: 25.1812x; 1.0007x over previous
import functools

import jax
import jax.numpy as jnp
from jax import lax
from jax.experimental import pallas as pl
from jax.experimental.pallas import tpu as pltpu
from jax.experimental.pallas import tpu_sc as plsc

N = 10000
E = 160000
D = 256
H = D // 2
NC = 2
NS = 16
N_PAD = 10240

_K = 80
_EPT = E // NS
_CHUNKS = _EPT // _K
_ZPT = N_PAD // NS
_RPT = 640
_RSTRIDE = 624
_RQ = _RPT // _K
_NBUF = 3

_sc_mesh = plsc.VectorSubcoreMesh(core_axis_name="c", subcore_axis_name="s")


@functools.partial(
    pl.kernel,
    out_type=[
        jax.ShapeDtypeStruct((NC * N, H), jnp.float32),
        jax.ShapeDtypeStruct((NC * N, H), jnp.float32),
    ],
    mesh=_sc_mesh,
    scratch_types=[
        pltpu.VMEM((_K,), jnp.float32),
        pltpu.VMEM((_ZPT,), jnp.float32),
        pltpu.VMEM((_RPT + 16,), jnp.float32),
        pltpu.VMEM((_EPT,), jnp.int32),
        pltpu.VMEM((_NBUF, _K), jnp.int32),
        pltpu.VMEM((_NBUF, _K, H), jnp.float32),
        pltpu.VMEM_SHARED((N_PAD,), jnp.float32),
        pltpu.VMEM_SHARED((N, H), jnp.float32),
    ] + [pltpu.SemaphoreType.DMA] * (3 * _NBUF),
)
def _gcn_sc_kernel(x_hbm, ei_hbm, t_hbm, z_hbm,
                   ones_v, zeros_v, tb, sbuf, didx, rows_v, deg_s, acc_s,
                   *sems):
    dsem = sems[:_NBUF]
    gsem = sems[_NBUF:2 * _NBUF]
    ssem = sems[2 * _NBUF:]
    c = lax.axis_index("c")
    s = lax.axis_index("s")
    row0 = s * _RSTRIDE
    ebase0 = s * _EPT

    for j in range(_K // 16):
        ones_v[pl.ds(j * 16, 16)] = jnp.ones((16,), jnp.float32)

    def zfill(i, carry):
        zeros_v[pl.ds(i * 16, 16)] = jnp.zeros((16,), jnp.float32)
        return carry

    lax.fori_loop(0, _ZPT // 16, zfill, 0)
    pltpu.sync_copy(zeros_v, deg_s.at[pl.ds(s * _ZPT, _ZPT)])
    pltpu.sync_copy(ei_hbm.at[pl.ds(ebase0, _EPT)], sbuf)
    off = c * N

    def adj(i, carry):
        sl = pl.ds(i * 16, 16)
        sbuf[sl] = sbuf[sl] + off
        return carry

    lax.fori_loop(0, _EPT // 16, adj, 0)
    plsc.subcore_barrier()

    def _didx(g, j):
        pltpu.async_copy(ei_hbm.at[pl.ds(E + ebase0 + g * _K, _K)],
                         didx.at[j], dsem[j])

    def _didx_wait(j):
        pltpu.make_async_copy(ei_hbm.at[pl.ds(0, _K)], didx.at[j],
                              dsem[j]).wait()

    def _hist(j):
        pltpu.async_copy(ones_v, deg_s.at[didx.at[j]], ssem[j], add=True)

    def _hist_wait(j):
        pltpu.make_async_copy(ones_v, deg_s.at[didx.at[0]], ssem[j]).wait()

    for j in range(_NBUF):
        _didx(j, j)
    nh = (_CHUNKS + _NBUF - 1) // _NBUF

    def hloop(k, carry):
        for j in range(_NBUF):
            g = k * _NBUF + j

            @pl.when(g < _CHUNKS)
            def _():
                _didx_wait(j)
                _hist(j)
            g2 = (k + 1) * _NBUF + j

            @pl.when(g2 < _CHUNKS)
            def _():
                _hist_wait(j)
                _didx(g2, j)
        return carry

    lax.fori_loop(0, nh, hloop, 0)
    for j in range(_NBUF):
        _hist_wait(j)
    plsc.subcore_barrier()

    pltpu.sync_copy(deg_s.at[pl.ds(row0, _RPT)], tb.at[pl.ds(0, _RPT)])

    def newt(i, carry):
        sl = pl.ds(i * 16, 16)
        d = tb[sl] + 1.0
        iv = lax.bitcast_convert_type(d, jnp.int32)
        iv = 0x5F3759DF - lax.shift_right_arithmetic(iv, 1)
        y = lax.bitcast_convert_type(iv, jnp.float32)
        for _ in range(3):
            y = y * (1.5 - 0.5 * d * y * y)
        tb[sl] = y
        return carry

    lax.fori_loop(0, _RPT // 16, newt, 0)

    def _scale_rows(j, qbase):
        lane0 = jnp.zeros((16,), jnp.int32)

        def srow(r, carry):
            dvec = tb[pl.ds(qbase + r, 16)]
            dv = dvec.at[lane0].get(mode="promise_in_bounds")
            for w in range(H // 16):
                sl = pl.ds(w * 16, 16)
                rows_v[j, r, sl] = rows_v[j, r, sl] * dv
            return carry

        lax.fori_loop(0, _K, srow, 0)

    def _zphase(colref):
        def _xload(q, j):
            pltpu.async_copy(colref.at[pl.ds(row0 + q * _K, _K)],
                             rows_v.at[j], gsem[j])

        def _xload_wait(j):
            pltpu.make_async_copy(colref.at[pl.ds(row0, _K)],
                                  rows_v.at[j], gsem[j]).wait()

        def _zw_wait(j):
            pltpu.make_async_copy(rows_v.at[j], z_hbm.at[pl.ds(0, _K)],
                                  ssem[j]).wait()
            pltpu.make_async_copy(rows_v.at[j], acc_s.at[pl.ds(0, _K)],
                                  dsem[j]).wait()

        _xload(0, 0)
        for q in range(_RQ):
            j = q & 1
            rq = row0 + q * _K
            _xload_wait(j)
            _scale_rows(j, q * _K)
            pltpu.async_copy(rows_v.at[j], z_hbm.at[pl.ds(c * N + rq, _K)],
                             ssem[j])
            pltpu.async_copy(rows_v.at[j], acc_s.at[pl.ds(rq, _K)], dsem[j])
            if q + 1 < _RQ:
                jn = (q + 1) & 1
                if q + 1 >= 2:
                    _zw_wait(jn)
                _xload(q + 1, jn)
        _zw_wait(_RQ & 1)
        _zw_wait((_RQ - 1) & 1)

    @pl.when(c == 0)
    def _():
        _zphase(x_hbm.at[:, pl.ds(0, H)])

    @pl.when(c == 1)
    def _():
        _zphase(x_hbm.at[:, pl.ds(H, H)])

    plsc.subcore_barrier()

    def _start(g, j):
        pltpu.async_copy(ei_hbm.at[pl.ds(E + ebase0 + g * _K, _K)],
                         didx.at[j], dsem[j])
        pltpu.async_copy(z_hbm.at[sbuf.at[pl.ds(g * _K, _K)]],
                         rows_v.at[j], gsem[j])

    def _start_wait(j):
        pltpu.make_async_copy(ei_hbm.at[pl.ds(0, _K)], didx.at[j],
                              dsem[j]).wait()
        pltpu.make_async_copy(z_hbm.at[sbuf.at[pl.ds(0, _K)]],
                              rows_v.at[j], gsem[j]).wait()

    def _scat(j):
        pltpu.async_copy(rows_v.at[j], acc_s.at[didx.at[j]], ssem[j], add=True)

    def _scat_wait(j):
        pltpu.make_async_copy(rows_v.at[j], acc_s.at[didx.at[0]],
                              ssem[j]).wait()

    for j in range(_NBUF):
        _start(j, j)
    ntri = (_CHUNKS + _NBUF - 1) // _NBUF

    def tri(k, carry):
        for j in range(_NBUF):
            g = k * _NBUF + j

            @pl.when(g < _CHUNKS)
            def _():
                _start_wait(j)
                _scat(j)
            g2 = (k + 1) * _NBUF + j

            @pl.when(g2 < _CHUNKS)
            def _():
                _scat_wait(j)
                _start(g2, j)
        return carry

    lax.fori_loop(0, ntri, tri, 0)
    for j in range(_NBUF):
        _scat_wait(j)
    plsc.subcore_barrier()

    def _aload(q, j):
        pltpu.async_copy(acc_s.at[pl.ds(row0 + q * _K, _K)],
                         rows_v.at[j], gsem[j])

    def _aload_wait(j):
        pltpu.make_async_copy(acc_s.at[pl.ds(row0, _K)], rows_v.at[j],
                              gsem[j]).wait()

    def _tw_wait(j):
        pltpu.make_async_copy(rows_v.at[j], t_hbm.at[pl.ds(0, _K)],
                              ssem[j]).wait()

    _aload(0, 0)
    for q in range(_RQ):
        j = q & 1
        rq = row0 + q * _K
        _aload_wait(j)
        _scale_rows(j, q * _K)
        pltpu.async_copy(rows_v.at[j], t_hbm.at[pl.ds(c * N + rq, _K)],
                         ssem[j])
        if q + 1 < _RQ:
            jn = (q + 1) & 1
            if q + 1 >= 2:
                _tw_wait(jn)
            _aload(q + 1, jn)
    _tw_wait(_RQ & 1)
    _tw_wait((_RQ - 1) & 1)


_BR = 2000


def _weights_body(wg_ref, wl_ref, bg_ref, bl_ref, wc_ref, c_ref):
    wl = wl_ref[...]
    wc_ref[...] = jnp.dot(wg_ref[...], wl, preferred_element_type=jnp.float32)
    c_ref[...] = (jnp.dot(bg_ref[...], wl, preferred_element_type=jnp.float32)
                  + bl_ref[...])


def _out_body(t0_ref, t1_ref, wc_ref, c_ref, o_ref):
    m = jnp.concatenate([t0_ref[...], t1_ref[...]], axis=1)
    o_ref[...] = (jnp.dot(m, wc_ref[...], preferred_element_type=jnp.float32)
                  + c_ref[...])


def kernel(x, edge_index, W_gcn, b_gcn, W_lin, b_lin):
    ei = edge_index.astype(jnp.int32)

    t, _ = _gcn_sc_kernel(x, ei.reshape(-1))

    wc, cvec = pl.pallas_call(
        _weights_body,
        out_shape=[
            jax.ShapeDtypeStruct((D, D), jnp.float32),
            jax.ShapeDtypeStruct((1, D), jnp.float32),
        ],
    )(W_gcn, W_lin, b_gcn.reshape(1, D), b_lin.reshape(1, D))

    nb = N // _BR
    out = pl.pallas_call(
        _out_body,
        grid=(nb,),
        in_specs=[
            pl.BlockSpec((_BR, H), lambda i: (i, 0)),
            pl.BlockSpec((_BR, H), lambda i: (i + nb, 0)),
            pl.BlockSpec((D, D), lambda i: (0, 0)),
            pl.BlockSpec((1, D), lambda i: (0, 0)),
        ],
        out_specs=pl.BlockSpec((_BR, D), lambda i: (i, 0)),
        out_shape=jax.ShapeDtypeStruct((N, D), jnp.float32),
    )(t, t, wc, cvec)

    return out
